# baseline (speedup 1.0000x reference)
import functools

import jax
import jax.numpy as jnp
from jax.experimental import pallas as pl
from jax.experimental.pallas import tpu as pltpu

K_ORDER = 25


def _cheb_kernel(L_ref, X_ref, out_ref, t1_ref, t2_ref):
    k = pl.program_id(0)

    @pl.when(k == 0)
    def _():
        T = X_ref[...]
        out_ref[0] = T
        t2_ref[...] = T

    @pl.when(k == 1)
    def _():
        T = jnp.dot(L_ref[...].astype(jnp.bfloat16),
                    X_ref[...].astype(jnp.bfloat16),
                    preferred_element_type=jnp.float32)
        out_ref[0] = T
        t1_ref[...] = T

    @pl.when(k >= 2)
    def _():
        T = 2.0 * jnp.dot(L_ref[...].astype(jnp.bfloat16),
                          t1_ref[...].astype(jnp.bfloat16),
                          preferred_element_type=jnp.float32) - t2_ref[...]
        out_ref[0] = T
        t2_ref[...] = t1_ref[...]
        t1_ref[...] = T


def _cheb_basis(L, X):
    N, C = X.shape
    return pl.pallas_call(
        _cheb_kernel,
        grid=(K_ORDER,),
        in_specs=[
            pl.BlockSpec((N, N), lambda k: (0, 0)),
            pl.BlockSpec((N, C), lambda k: (0, 0)),
        ],
        out_specs=pl.BlockSpec((1, N, C), lambda k: (k, 0, 0)),
        out_shape=jax.ShapeDtypeStruct((K_ORDER, N, C), jnp.float32),
        scratch_shapes=[
            pltpu.VMEM((N, C), jnp.float32),
            pltpu.VMEM((N, C), jnp.float32),
        ],
    )(L, X)


def _linear_kernel(A_ref, W_ref, b_ref, out_ref, *, relu):
    h = jnp.dot(A_ref[...], W_ref[...], preferred_element_type=jnp.float32)
    h = h + b_ref[...]
    if relu:
        h = jnp.maximum(h, 0.0)
    out_ref[...] = h


def _linear_relu(A, W, b, block_m=3584):
    M, F = A.shape
    G = W.shape[1]
    grid = M // block_m
    return pl.pallas_call(
        functools.partial(_linear_kernel, relu=True),
        grid=(grid,),
        in_specs=[
            pl.BlockSpec((block_m, F), lambda i: (i, 0)),
            pl.BlockSpec((F, G), lambda i: (0, 0)),
            pl.BlockSpec((1, G), lambda i: (0, 0)),
        ],
        out_specs=pl.BlockSpec((block_m, G), lambda i: (i, 0)),
        out_shape=jax.ShapeDtypeStruct((M, G), jnp.float32),
    )(A, W, b.reshape(1, G))


def _fc_kernel(h_ref, W1_ref, b1_ref, W2_ref, b2_ref, out_ref):
    h1 = jnp.dot(h_ref[...], W1_ref[...], preferred_element_type=jnp.float32)
    h1 = jnp.maximum(h1 + b1_ref[...], 0.0)
    h2 = jnp.dot(h1, W2_ref[...], preferred_element_type=jnp.float32)
    h2 = h2 + b2_ref[...]
    m = jnp.max(h2, axis=1, keepdims=True)
    lse = jnp.log(jnp.sum(jnp.exp(h2 - m), axis=1, keepdims=True)) + m
    out_ref[...] = h2 - lse


def _fc_head(h, fc1W, fc1b, fc2W, fc2b):
    B, D = h.shape
    H1 = fc1W.shape[1]
    G = fc2W.shape[1]
    return pl.pallas_call(
        _fc_kernel,
        in_specs=[
            pl.BlockSpec((B, D), lambda: (0, 0)),
            pl.BlockSpec((D, H1), lambda: (0, 0)),
            pl.BlockSpec((1, H1), lambda: (0, 0)),
            pl.BlockSpec((H1, G), lambda: (0, 0)),
            pl.BlockSpec((1, G), lambda: (0, 0)),
        ],
        out_specs=pl.BlockSpec((B, G), lambda: (0, 0)),
        out_shape=jax.ShapeDtypeStruct((B, G), jnp.float32),
    )(h, fc1W, fc1b.reshape(1, H1), fc2W, fc2b.reshape(1, G))


def _gcn_layer(L, X, W, b, fin):
    N = X.shape[0]
    B = X.shape[1] // fin
    fout = W.shape[1]
    Xs = _cheb_basis(L, X)
    A = (Xs.reshape(K_ORDER, N, B, fin)
         .transpose(1, 2, 3, 0)
         .reshape(N * B, fin * K_ORDER))
    H = _linear_relu(A, W, b)
    return H.reshape(N, B * fout)


def kernel(x, L, W1, b1, W2, b2, W3, b3, fc1W, fc1b, fc2W, fc2b):
    B, N, _ = x.shape
    X = x[:, :, 0].T
    H1 = _gcn_layer(L, X, W1, b1, 1)
    H2 = _gcn_layer(L, H1, W2, b2, 30)
    H3 = _gcn_layer(L, H2, W3, b3, 20)
    Hf = H3.reshape(N, B, 10).transpose(1, 0, 2).reshape(B, N * 10)
    return _fc_head(Hf, fc1W, fc1b, fc2W, fc2b)


# speedup vs baseline: 4.3061x; 4.3061x over previous
import jax
import jax.numpy as jnp
from jax.experimental import pallas as pl
from jax.experimental.pallas import tpu as pltpu

K_ORDER = 25


def _layer1_kernel(xT_ref, Lt_ref, W_ref, b_ref, out_ref, s1_ref, s2_ref,
                   stack_ref, *, n_batch):
    k = pl.program_id(0)

    @pl.when(k == 0)
    def _():
        S = xT_ref[...]
        stack_ref[pl.ds(0, 1)] = S[None]
        s2_ref[...] = S

    @pl.when(k == 1)
    def _():
        S = jnp.dot(s2_ref[...], Lt_ref[...], preferred_element_type=jnp.float32)
        stack_ref[pl.ds(1, 1)] = S[None]
        s1_ref[...] = S

    @pl.when(k >= 2)
    def _():
        S = 2.0 * jnp.dot(s1_ref[...], Lt_ref[...],
                          preferred_element_type=jnp.float32) - s2_ref[...]
        stack_ref[pl.ds(k, 1)] = S[None]
        s2_ref[...] = s1_ref[...]
        s1_ref[...] = S

    @pl.when(k == K_ORDER - 1)
    def _():
        W = W_ref[...]
        b = b_ref[...]
        for bi in range(n_batch):
            slab = stack_ref[:, bi, :]
            h = jnp.dot(W, slab, preferred_element_type=jnp.float32)
            out_ref[bi] = jnp.maximum(h + b, 0.0)


def _layer1(xT, Lt, W1pT, b1col):
    B, N = xT.shape
    Gp = W1pT.shape[0]
    return pl.pallas_call(
        lambda *refs: _layer1_kernel(*refs, n_batch=B),
        grid=(K_ORDER,),
        in_specs=[
            pl.BlockSpec((B, N), lambda k: (0, 0)),
            pl.BlockSpec((N, N), lambda k: (0, 0)),
            pl.BlockSpec((Gp, K_ORDER), lambda k: (0, 0)),
            pl.BlockSpec((Gp, 1), lambda k: (0, 0)),
        ],
        out_specs=pl.BlockSpec((B, Gp, N), lambda k: (0, 0, 0)),
        out_shape=jax.ShapeDtypeStruct((B, Gp, N), jnp.float32),
        scratch_shapes=[
            pltpu.VMEM((B, N), jnp.float32),
            pltpu.VMEM((B, N), jnp.float32),
            pltpu.VMEM((K_ORDER, B, N), jnp.float32),
        ],
    )(xT, Lt, W1pT, b1col)


def _pre_kernel(H_ref, W_ref, out_ref, *, gp):
    A = jnp.dot(W_ref[...], H_ref[0], preferred_element_type=jnp.float32)
    out_ref[:, 0, :, :] = A.reshape(K_ORDER, gp, A.shape[-1])


def _precompute(H, Wall, Gp):
    B, Fp, N = H.shape
    return pl.pallas_call(
        lambda *refs: _pre_kernel(*refs, gp=Gp),
        grid=(B,),
        in_specs=[
            pl.BlockSpec((1, Fp, N), lambda b: (b, 0, 0)),
            pl.BlockSpec((K_ORDER * Gp, Fp), lambda b: (0, 0)),
        ],
        out_specs=pl.BlockSpec((K_ORDER, 1, Gp, N), lambda b: (0, b, 0, 0)),
        out_shape=jax.ShapeDtypeStruct((K_ORDER, B, Gp, N), jnp.float32),
    )(H, Wall)


def _clenshaw_kernel(Lt_ref, a_ref, b_ref, out_ref, c1_ref, c2_ref,
                     *, n_batch, gp):
    i = pl.program_id(0)
    N = Lt_ref.shape[0]
    rows = n_batch * gp
    a = a_ref[0].reshape(rows, N)

    @pl.when(i == 0)
    def _():
        c1_ref[...] = a
        c2_ref[...] = jnp.zeros_like(c2_ref)

    @pl.when((i >= 1) & (i <= K_ORDER - 2))
    def _():
        bnew = 2.0 * jnp.dot(c1_ref[...], Lt_ref[...],
                             preferred_element_type=jnp.float32) - c2_ref[...] + a
        c2_ref[...] = c1_ref[...]
        c1_ref[...] = bnew

    @pl.when(i == K_ORDER - 1)
    def _():
        Hf = jnp.dot(c1_ref[...], Lt_ref[...],
                     preferred_element_type=jnp.float32) - c2_ref[...] + a
        H = Hf.reshape(n_batch, gp, N) + b_ref[...][None]
        out_ref[...] = jnp.maximum(H, 0.0)


def _clenshaw(Lt, PRE, bcol):
    K, B, Gp, N = PRE.shape
    return pl.pallas_call(
        lambda *refs: _clenshaw_kernel(*refs, n_batch=B, gp=Gp),
        grid=(K_ORDER,),
        in_specs=[
            pl.BlockSpec((N, N), lambda i: (0, 0)),
            pl.BlockSpec((1, B, Gp, N), lambda i: (K_ORDER - 1 - i, 0, 0, 0)),
            pl.BlockSpec((Gp, 1), lambda i: (0, 0)),
        ],
        out_specs=pl.BlockSpec((B, Gp, N), lambda i: (0, 0, 0)),
        out_shape=jax.ShapeDtypeStruct((B, Gp, N), jnp.float32),
        scratch_shapes=[
            pltpu.VMEM((B * Gp, N), jnp.float32),
            pltpu.VMEM((B * Gp, N), jnp.float32),
        ],
    )(Lt, PRE, bcol)


def _fc_kernel(H_ref, W1_ref, b1_ref, W2_ref, b2_ref, out_ref):
    acc = b1_ref[...]
    for c in range(10):
        acc = acc + jnp.dot(H_ref[:, c, :], W1_ref[:, c, :],
                            preferred_element_type=jnp.float32)
    h1 = jnp.maximum(acc, 0.0)
    h2 = jnp.dot(h1, W2_ref[...], preferred_element_type=jnp.float32)
    h2 = h2 + b2_ref[...]
    m = jnp.max(h2, axis=1, keepdims=True)
    lse = jnp.log(jnp.sum(jnp.exp(h2 - m), axis=1, keepdims=True)) + m
    out_ref[...] = h2 - lse


def _fc_head(H3, fc1Wr, fc1b, fc2W, fc2b):
    B, Cp, N = H3.shape
    H1dim = fc1Wr.shape[-1]
    G = fc2W.shape[1]
    return pl.pallas_call(
        _fc_kernel,
        in_specs=[
            pl.BlockSpec((B, Cp, N), lambda: (0, 0, 0)),
            pl.BlockSpec((N, 10, H1dim), lambda: (0, 0, 0)),
            pl.BlockSpec((1, H1dim), lambda: (0, 0)),
            pl.BlockSpec((H1dim, G), lambda: (0, 0)),
            pl.BlockSpec((1, G), lambda: (0, 0)),
        ],
        out_specs=pl.BlockSpec((B, G), lambda: (0, 0)),
        out_shape=jax.ShapeDtypeStruct((B, G), jnp.float32),
    )(H3, fc1Wr, fc1b.reshape(1, H1dim), fc2W, fc2b.reshape(1, G))


def kernel(x, L, W1, b1, W2, b2, W3, b3, fc1W, fc1b, fc2W, fc2b):
    B, N, _ = x.shape
    K = K_ORDER
    xT = x[:, :, 0]
    Lt = L.T

    G1p = 32
    W1pT = jnp.zeros((G1p, K), jnp.float32).at[:W1.shape[1], :].set(W1.T)
    b1col = jnp.zeros((G1p, 1), jnp.float32).at[:b1.shape[0], 0].set(b1)

    F2, G2 = 30, 20
    F2p, G2p = 32, 24
    W2r = W2.reshape(F2, K, G2).transpose(1, 2, 0)
    Wall2 = (jnp.zeros((K, G2p, F2p), jnp.float32)
             .at[:, :G2, :F2].set(W2r).reshape(K * G2p, F2p))
    b2col = jnp.zeros((G2p, 1), jnp.float32).at[:G2, 0].set(b2)

    F3, G3 = 20, 10
    F3p, G3p = 24, 16
    W3r = W3.reshape(F3, K, G3).transpose(1, 2, 0)
    Wall3 = (jnp.zeros((K, G3p, F3p), jnp.float32)
             .at[:, :G3, :F3].set(W3r).reshape(K * G3p, F3p))
    b3col = jnp.zeros((G3p, 1), jnp.float32).at[:G3, 0].set(b3)

    fc1Wr = fc1W.reshape(N, 10, fc1W.shape[1])

    H1 = _layer1(xT, Lt, W1pT, b1col)
    PRE2 = _precompute(H1, Wall2, G2p)
    H2 = _clenshaw(Lt, PRE2, b2col)
    PRE3 = _precompute(H2, Wall3, G3p)
    H3 = _clenshaw(Lt, PRE3, b3col)
    return _fc_head(H3, fc1Wr, fc1b, fc2W, fc2b)


# speedup vs baseline: 5.5392x; 1.2863x over previous
import jax
import jax.numpy as jnp
from jax.experimental import pallas as pl
from jax.experimental.pallas import tpu as pltpu

K_ORDER = 25
BCHUNK = 8


def _layer1_kernel(xT_ref, Lt_ref, W_ref, b_ref, out_ref, s1_ref, s2_ref,
                   stack_ref, *, n_batch):
    k = pl.program_id(0)

    @pl.when(k == 0)
    def _():
        S = xT_ref[...]
        stack_ref[pl.ds(0, 1)] = S[None]
        s2_ref[...] = S

    @pl.when(k == 1)
    def _():
        S = jnp.dot(s2_ref[...], Lt_ref[...], preferred_element_type=jnp.float32)
        stack_ref[pl.ds(1, 1)] = S[None]
        s1_ref[...] = S

    @pl.when(k >= 2)
    def _():
        S = 2.0 * jnp.dot(s1_ref[...], Lt_ref[...],
                          preferred_element_type=jnp.float32) - s2_ref[...]
        stack_ref[pl.ds(k, 1)] = S[None]
        s2_ref[...] = s1_ref[...]
        s1_ref[...] = S

    @pl.when(k == K_ORDER - 1)
    def _():
        W = W_ref[...]
        b = b_ref[...]
        for bi in range(n_batch):
            slab = stack_ref[:, bi, :]
            h = jnp.dot(W, slab, preferred_element_type=jnp.float32)
            out_ref[:, bi, :] = jnp.maximum(h + b, 0.0)


def _layer1(xT, Lt, W1T, b1col):
    B, N = xT.shape
    G = W1T.shape[0]
    return pl.pallas_call(
        lambda *refs: _layer1_kernel(*refs, n_batch=B),
        grid=(K_ORDER,),
        in_specs=[
            pl.BlockSpec((B, N), lambda k: (0, 0)),
            pl.BlockSpec((N, N), lambda k: (0, 0)),
            pl.BlockSpec((G, K_ORDER), lambda k: (0, 0)),
            pl.BlockSpec((G, 1), lambda k: (0, 0)),
        ],
        out_specs=pl.BlockSpec((G, B, N), lambda k: (0, 0, 0)),
        out_shape=jax.ShapeDtypeStruct((G, B, N), jnp.float32),
        scratch_shapes=[
            pltpu.VMEM((B, N), jnp.float32),
            pltpu.VMEM((B, N), jnp.float32),
            pltpu.VMEM((K_ORDER, B, N), jnp.float32),
        ],
    )(xT, Lt, W1T, b1col)


def _pre_kernel(H_ref, W_ref, out_ref, *, g_out):
    for bi in range(BCHUNK):
        h = H_ref[:, bi, :]
        A = jnp.dot(W_ref[...], h, preferred_element_type=jnp.float32)
        out_ref[:, :, bi, :] = A.reshape(K_ORDER, g_out, A.shape[-1])


def _precompute(H, Wall, Gout):
    F, B, N = H.shape
    return pl.pallas_call(
        lambda *refs: _pre_kernel(*refs, g_out=Gout),
        grid=(B // BCHUNK,),
        in_specs=[
            pl.BlockSpec((F, BCHUNK, N), lambda c: (0, c, 0)),
            pl.BlockSpec((K_ORDER * Gout, F), lambda c: (0, 0)),
        ],
        out_specs=pl.BlockSpec((K_ORDER, Gout, BCHUNK, N), lambda c: (0, 0, c, 0)),
        out_shape=jax.ShapeDtypeStruct((K_ORDER, Gout, B, N), jnp.float32),
    )(H, Wall)


def _clenshaw_kernel(Lt_ref, a_ref, b_ref, out_ref, c1_ref, c2_ref,
                     *, n_batch, g_out):
    i = pl.program_id(0)
    N = Lt_ref.shape[0]
    rows = n_batch * g_out
    a = a_ref[0].reshape(rows, N)

    @pl.when(i == 0)
    def _():
        c1_ref[...] = a
        c2_ref[...] = jnp.zeros_like(c2_ref)

    @pl.when((i >= 1) & (i <= K_ORDER - 2))
    def _():
        bnew = 2.0 * jnp.dot(c1_ref[...], Lt_ref[...],
                             preferred_element_type=jnp.float32) - c2_ref[...] + a
        c2_ref[...] = c1_ref[...]
        c1_ref[...] = bnew

    @pl.when(i == K_ORDER - 1)
    def _():
        Hf = jnp.dot(c1_ref[...], Lt_ref[...],
                     preferred_element_type=jnp.float32) - c2_ref[...] + a
        H = Hf.reshape(g_out, n_batch, N) + b_ref[...][:, None]
        out_ref[...] = jnp.maximum(H, 0.0)


def _clenshaw(Lt, PRE, bcol):
    K, Gout, B, N = PRE.shape
    return pl.pallas_call(
        lambda *refs: _clenshaw_kernel(*refs, n_batch=B, g_out=Gout),
        grid=(K_ORDER,),
        in_specs=[
            pl.BlockSpec((N, N), lambda i: (0, 0)),
            pl.BlockSpec((1, Gout, B, N), lambda i: (K_ORDER - 1 - i, 0, 0, 0)),
            pl.BlockSpec((Gout, 1), lambda i: (0, 0)),
        ],
        out_specs=pl.BlockSpec((Gout, B, N), lambda i: (0, 0, 0)),
        out_shape=jax.ShapeDtypeStruct((Gout, B, N), jnp.float32),
        scratch_shapes=[
            pltpu.VMEM((Gout * B, N), jnp.float32),
            pltpu.VMEM((Gout * B, N), jnp.float32),
        ],
    )(Lt, PRE, bcol)


def _fc_kernel(H_ref, W1_ref, b1_ref, W2_ref, b2_ref, out_ref):
    acc = b1_ref[...]
    for c in range(10):
        acc = acc + jnp.dot(H_ref[c], W1_ref[:, c, :],
                            preferred_element_type=jnp.float32)
    h1 = jnp.maximum(acc, 0.0)
    h2 = jnp.dot(h1, W2_ref[...], preferred_element_type=jnp.float32)
    h2 = h2 + b2_ref[...]
    m = jnp.max(h2, axis=1, keepdims=True)
    lse = jnp.log(jnp.sum(jnp.exp(h2 - m), axis=1, keepdims=True)) + m
    out_ref[...] = h2 - lse


def _fc_head(H3, fc1Wr, fc1b, fc2W, fc2b):
    C, B, N = H3.shape
    H1dim = fc1Wr.shape[-1]
    G = fc2W.shape[1]
    return pl.pallas_call(
        _fc_kernel,
        in_specs=[
            pl.BlockSpec((C, B, N), lambda: (0, 0, 0)),
            pl.BlockSpec((N, 10, H1dim), lambda: (0, 0, 0)),
            pl.BlockSpec((1, H1dim), lambda: (0, 0)),
            pl.BlockSpec((H1dim, G), lambda: (0, 0)),
            pl.BlockSpec((1, G), lambda: (0, 0)),
        ],
        out_specs=pl.BlockSpec((B, G), lambda: (0, 0)),
        out_shape=jax.ShapeDtypeStruct((B, G), jnp.float32),
    )(H3, fc1Wr, fc1b.reshape(1, H1dim), fc2W, fc2b.reshape(1, G))


def kernel(x, L, W1, b1, W2, b2, W3, b3, fc1W, fc1b, fc2W, fc2b):
    B, N, _ = x.shape
    K = K_ORDER
    xT = x[:, :, 0]
    Lt = L

    W1T = W1.T
    b1col = b1[:, None]

    F2, G2 = 30, 20
    W2r = W2.reshape(F2, K, G2).transpose(1, 2, 0)
    Wall2 = W2r.reshape(K * G2, F2)
    b2col = b2[:, None]

    F3, G3 = 20, 10
    W3r = W3.reshape(F3, K, G3).transpose(1, 2, 0)
    Wall3 = W3r.reshape(K * G3, F3)
    b3col = b3[:, None]

    fc1Wr = fc1W.reshape(N, 10, fc1W.shape[1])

    H1 = _layer1(xT, Lt, W1T, b1col)
    PRE2 = _precompute(H1, Wall2, G2)
    H2 = _clenshaw(Lt, PRE2, b2col)
    PRE3 = _precompute(H2, Wall3, G3)
    H3 = _clenshaw(Lt, PRE3, b3col)
    return _fc_head(H3, fc1Wr, fc1b, fc2W, fc2b)


# speedup vs baseline: 5.5904x; 1.0093x over previous
import jax
import jax.numpy as jnp
from jax.experimental import pallas as pl
from jax.experimental.pallas import tpu as pltpu

K_ORDER = 25
BCHUNK = 8


def _layer1_kernel(xT_ref, Lt_ref, W_ref, b_ref, Wall_ref, out_ref,
                   s1_ref, s2_ref, stack_ref, *, g_out):
    i = pl.program_id(0)

    @pl.when(i == 0)
    def _():
        S = xT_ref[...]
        stack_ref[pl.ds(0, 1)] = S[None]
        s2_ref[...] = S

    @pl.when(i == 1)
    def _():
        S = jnp.dot(s2_ref[...], Lt_ref[...], preferred_element_type=jnp.float32)
        stack_ref[pl.ds(1, 1)] = S[None]
        s1_ref[...] = S

    @pl.when((i >= 2) & (i < K_ORDER))
    def _():
        S = 2.0 * jnp.dot(s1_ref[...], Lt_ref[...],
                          preferred_element_type=jnp.float32) - s2_ref[...]
        stack_ref[pl.ds(i, 1)] = S[None]
        s2_ref[...] = s1_ref[...]
        s1_ref[...] = S

    @pl.when(i >= K_ORDER)
    def _():
        W = W_ref[...]
        b = b_ref[...]
        base = (i - K_ORDER) * BCHUNK
        chunk = stack_ref[:, pl.ds(base, BCHUNK), :]
        for bi in range(BCHUNK):
            slab = chunk[:, bi, :]
            h = jnp.maximum(jnp.dot(W, slab,
                                    preferred_element_type=jnp.float32) + b, 0.0)
            A = jnp.dot(Wall_ref[...], h, preferred_element_type=jnp.float32)
            out_ref[:, :, bi, :] = A.reshape(K_ORDER, g_out, A.shape[-1])


def _layer1_pre2(xT, Lt, W1T, b1col, Wall2, Gout):
    B, N = xT.shape
    G = W1T.shape[0]
    return pl.pallas_call(
        lambda *refs: _layer1_kernel(*refs, g_out=Gout),
        grid=(K_ORDER + B // BCHUNK,),
        in_specs=[
            pl.BlockSpec((B, N), lambda i: (0, 0)),
            pl.BlockSpec((N, N), lambda i: (0, 0)),
            pl.BlockSpec((G, K_ORDER), lambda i: (0, 0)),
            pl.BlockSpec((G, 1), lambda i: (0, 0)),
            pl.BlockSpec((K_ORDER * Gout, G), lambda i: (0, 0)),
        ],
        out_specs=pl.BlockSpec(
            (K_ORDER, Gout, BCHUNK, N),
            lambda i: (0, 0, jnp.maximum(i - K_ORDER, 0), 0)),
        out_shape=jax.ShapeDtypeStruct((K_ORDER, Gout, B, N), jnp.float32),
        scratch_shapes=[
            pltpu.VMEM((B, N), jnp.float32),
            pltpu.VMEM((B, N), jnp.float32),
            pltpu.VMEM((K_ORDER, B, N), jnp.float32),
        ],
    )(xT, Lt, W1T, b1col, Wall2)


def _pre_kernel(H_ref, W_ref, out_ref, *, g_out):
    for bi in range(BCHUNK):
        h = H_ref[:, bi, :]
        A = jnp.dot(W_ref[...], h, preferred_element_type=jnp.float32)
        out_ref[:, :, bi, :] = A.reshape(K_ORDER, g_out, A.shape[-1])


def _precompute(H, Wall, Gout):
    F, B, N = H.shape
    return pl.pallas_call(
        lambda *refs: _pre_kernel(*refs, g_out=Gout),
        grid=(B // BCHUNK,),
        in_specs=[
            pl.BlockSpec((F, BCHUNK, N), lambda c: (0, c, 0)),
            pl.BlockSpec((K_ORDER * Gout, F), lambda c: (0, 0)),
        ],
        out_specs=pl.BlockSpec((K_ORDER, Gout, BCHUNK, N), lambda c: (0, 0, c, 0)),
        out_shape=jax.ShapeDtypeStruct((K_ORDER, Gout, B, N), jnp.float32),
    )(H, Wall)


def _clenshaw_kernel(Lt_ref, a_ref, b_ref, out_ref, c1_ref, c2_ref,
                     *, n_batch, g_out):
    i = pl.program_id(0)
    N = Lt_ref.shape[0]
    rows = n_batch * g_out
    a = a_ref[0].reshape(rows, N)

    @pl.when(i == 0)
    def _():
        c1_ref[...] = a
        c2_ref[...] = jnp.zeros_like(c2_ref)

    @pl.when((i >= 1) & (i <= K_ORDER - 2))
    def _():
        bnew = 2.0 * jnp.dot(c1_ref[...], Lt_ref[...],
                             preferred_element_type=jnp.float32) - c2_ref[...] + a
        c2_ref[...] = c1_ref[...]
        c1_ref[...] = bnew

    @pl.when(i == K_ORDER - 1)
    def _():
        Hf = jnp.dot(c1_ref[...], Lt_ref[...],
                     preferred_element_type=jnp.float32) - c2_ref[...] + a
        H = Hf.reshape(g_out, n_batch, N) + b_ref[...][:, None]
        out_ref[...] = jnp.maximum(H, 0.0)


def _clenshaw(Lt, PRE, bcol):
    K, Gout, B, N = PRE.shape
    return pl.pallas_call(
        lambda *refs: _clenshaw_kernel(*refs, n_batch=B, g_out=Gout),
        grid=(K_ORDER,),
        in_specs=[
            pl.BlockSpec((N, N), lambda i: (0, 0)),
            pl.BlockSpec((1, Gout, B, N), lambda i: (K_ORDER - 1 - i, 0, 0, 0)),
            pl.BlockSpec((Gout, 1), lambda i: (0, 0)),
        ],
        out_specs=pl.BlockSpec((Gout, B, N), lambda i: (0, 0, 0)),
        out_shape=jax.ShapeDtypeStruct((Gout, B, N), jnp.float32),
        scratch_shapes=[
            pltpu.VMEM((Gout * B, N), jnp.float32),
            pltpu.VMEM((Gout * B, N), jnp.float32),
        ],
    )(Lt, PRE, bcol)


def _fc_kernel(H_ref, W1_ref, b1_ref, W2_ref, b2_ref, out_ref):
    acc = b1_ref[...]
    for c in range(10):
        acc = acc + jnp.dot(H_ref[c], W1_ref[:, c, :],
                            preferred_element_type=jnp.float32)
    h1 = jnp.maximum(acc, 0.0)
    h2 = jnp.dot(h1, W2_ref[...], preferred_element_type=jnp.float32)
    h2 = h2 + b2_ref[...]
    m = jnp.max(h2, axis=1, keepdims=True)
    lse = jnp.log(jnp.sum(jnp.exp(h2 - m), axis=1, keepdims=True)) + m
    out_ref[...] = h2 - lse


def _fc_head(H3, fc1Wr, fc1b, fc2W, fc2b):
    C, B, N = H3.shape
    H1dim = fc1Wr.shape[-1]
    G = fc2W.shape[1]
    return pl.pallas_call(
        _fc_kernel,
        in_specs=[
            pl.BlockSpec((C, B, N), lambda: (0, 0, 0)),
            pl.BlockSpec((N, 10, H1dim), lambda: (0, 0, 0)),
            pl.BlockSpec((1, H1dim), lambda: (0, 0)),
            pl.BlockSpec((H1dim, G), lambda: (0, 0)),
            pl.BlockSpec((1, G), lambda: (0, 0)),
        ],
        out_specs=pl.BlockSpec((B, G), lambda: (0, 0)),
        out_shape=jax.ShapeDtypeStruct((B, G), jnp.float32),
    )(H3, fc1Wr, fc1b.reshape(1, H1dim), fc2W, fc2b.reshape(1, G))


def kernel(x, L, W1, b1, W2, b2, W3, b3, fc1W, fc1b, fc2W, fc2b):
    B, N, _ = x.shape
    K = K_ORDER
    xT = x[:, :, 0]
    Lt = L

    W1T = W1.T
    b1col = b1[:, None]

    F2, G2 = 30, 20
    W2r = W2.reshape(F2, K, G2).transpose(1, 2, 0)
    Wall2 = W2r.reshape(K * G2, F2)
    b2col = b2[:, None]

    F3, G3 = 20, 10
    W3r = W3.reshape(F3, K, G3).transpose(1, 2, 0)
    Wall3 = W3r.reshape(K * G3, F3)
    b3col = b3[:, None]

    fc1Wr = fc1W.reshape(N, 10, fc1W.shape[1])

    PRE2 = _layer1_pre2(xT, Lt, W1T, b1col, Wall2, G2)
    H2 = _clenshaw(Lt, PRE2, b2col)
    PRE3 = _precompute(H2, Wall3, G3)
    H3 = _clenshaw(Lt, PRE3, b3col)
    return _fc_head(H3, fc1Wr, fc1b, fc2W, fc2b)
